# baseline (speedup 1.0000x reference)
import functools

import jax
import jax.numpy as jnp
from jax import lax
from jax.experimental import pallas as pl
from jax.experimental.pallas import tpu as pltpu
from jax.experimental.pallas import tpu_sc as plsc

B, T, D, E, V = 128, 32, 32, 256, 32
MASK_ID = 0
BD = B * D

NC, NS = 2, 16
NW = NC * NS
B_PER_W = BD // NW

G = 8


def _sc_gather_body(idx_hbm, et_hbm, gt_hbm, out_f, out_g, idx_v, rows_f, rows_g,
                    sem_f, sem_g):
    wid = lax.axis_index("s") * NC + lax.axis_index("c")
    base = wid * B_PER_W
    pltpu.sync_copy(idx_hbm.at[pl.ds(base, B_PER_W)], idx_v)
    cp_f = pltpu.async_copy(et_hbm.at[idx_v], rows_f, sem_f)
    cp_g = pltpu.async_copy(gt_hbm.at[idx_v], rows_g, sem_g)
    cp_f.wait()
    cp_g.wait()
    pltpu.sync_copy(rows_f, out_f.at[pl.ds(base, B_PER_W)])
    pltpu.sync_copy(rows_g, out_g.at[pl.ds(base, B_PER_W)])


_sc_gather = functools.partial(
    pl.kernel,
    mesh=plsc.VectorSubcoreMesh(core_axis_name="c", subcore_axis_name="s"),
    out_type=[
        jax.ShapeDtypeStruct((BD, E), jnp.float32),
        jax.ShapeDtypeStruct((BD, E), jnp.float32),
    ],
    scratch_types=[
        pltpu.VMEM((B_PER_W,), jnp.int32),
        pltpu.VMEM((B_PER_W, E), jnp.float32),
        pltpu.VMEM((B_PER_W, E), jnp.float32),
        pltpu.SemaphoreType.DMA,
        pltpu.SemaphoreType.DMA,
    ],
)(_sc_gather_body)


def _tc_body(ids_ref, pp_ref, ff_ref, gg_ref, w_ref, b_ref, out_ref, valid_ref):
    i = pl.program_id(0)
    ids = ids_ref[pl.ds(i * G, G), :]
    w = w_ref[0, :]
    bias = b_ref[0, :]
    for g in range(G):
        gamma = gg_ref[g]
        feat = ff_ref[g]
        a = gamma * w[None, :]
        c = gamma * bias[None, :] + feat
        pp = pp_ref[g]
        out_ref[g] = pp[:, :, None] * a[None] + c[None]
    valid_ref[...] = jnp.broadcast_to((ids != MASK_ID)[:, None, :], (G, T, D))


def kernel(proprio, dof_ids, embed_table, gamma_table, W, b):
    dof_ids = dof_ids.astype(jnp.int32)
    idx_flat = dof_ids.reshape(BD)
    feat_g, gamma_g = _sc_gather(idx_flat, embed_table, gamma_table)
    feat_g = feat_g.reshape(B, D, E)
    gamma_g = gamma_g.reshape(B, D, E)
    b2 = b.reshape(1, E)
    grid = (B // G,)
    tokens, valid = pl.pallas_call(
        _tc_body,
        grid=grid,
        in_specs=[
            pl.BlockSpec((B, D), lambda i: (0, 0)),
            pl.BlockSpec((G, T, D), lambda i: (i, 0, 0)),
            pl.BlockSpec((G, D, E), lambda i: (i, 0, 0)),
            pl.BlockSpec((G, D, E), lambda i: (i, 0, 0)),
            pl.BlockSpec((1, E), lambda i: (0, 0)),
            pl.BlockSpec((1, E), lambda i: (0, 0)),
        ],
        out_specs=[
            pl.BlockSpec((G, T, D, E), lambda i: (i, 0, 0, 0)),
            pl.BlockSpec((G, T, D), lambda i: (i, 0, 0)),
        ],
        out_shape=[
            jax.ShapeDtypeStruct((B, T, D, E), jnp.float32),
            jax.ShapeDtypeStruct((B, T, D), jnp.bool_),
        ],
    )(dof_ids, proprio, feat_g, gamma_g, W, b2)
    return tokens, valid


# speedup vs baseline: 1.1165x; 1.1165x over previous
import functools

import jax
import jax.numpy as jnp
from jax import lax
from jax.experimental import pallas as pl
from jax.experimental.pallas import tpu as pltpu
from jax.experimental.pallas import tpu_sc as plsc

B, T, D, E, V = 128, 32, 32, 256, 32
MASK_ID = 0

H = B // 2
HD = H * D

NC, NS = 2, 16
NW = NC * NS
B_PER_W = HD // NW

G = 8
HBLK = H // G


def _sc_gather_body(idx_hbm, et_hbm, gt_hbm, out_f, out_g, idx_v, rows_f, rows_g,
                    sem_f, sem_g):
    wid = lax.axis_index("s") * NC + lax.axis_index("c")
    base = wid * B_PER_W
    pltpu.sync_copy(idx_hbm.at[pl.ds(base, B_PER_W)], idx_v)
    cp_f = pltpu.async_copy(et_hbm.at[idx_v], rows_f, sem_f)
    cp_g = pltpu.async_copy(gt_hbm.at[idx_v], rows_g, sem_g)
    cp_f.wait()
    cp_g.wait()
    pltpu.sync_copy(rows_f, out_f.at[pl.ds(base, B_PER_W)])
    pltpu.sync_copy(rows_g, out_g.at[pl.ds(base, B_PER_W)])


_sc_gather = functools.partial(
    pl.kernel,
    mesh=plsc.VectorSubcoreMesh(core_axis_name="c", subcore_axis_name="s"),
    out_type=[
        jax.ShapeDtypeStruct((HD, E), jnp.float32),
        jax.ShapeDtypeStruct((HD, E), jnp.float32),
    ],
    scratch_types=[
        pltpu.VMEM((B_PER_W,), jnp.int32),
        pltpu.VMEM((B_PER_W, E), jnp.float32),
        pltpu.VMEM((B_PER_W, E), jnp.float32),
        pltpu.SemaphoreType.DMA,
        pltpu.SemaphoreType.DMA,
    ],
)(_sc_gather_body)


def _tc1_body(ids_ref, pp_ref, et_ref, gt_ref, w_ref, b_ref, out_ref, valid_ref):
    i = pl.program_id(0)
    ids = ids_ref[pl.ds(i * G, G), :]
    w = w_ref[0, :]
    bias = b_ref[0, :]
    for g in range(G):
        idsg = ids[g, :]
        onehot = (idsg[:, None] == lax.broadcasted_iota(jnp.int32, (D, V), 1)
                  ).astype(jnp.float32)
        gamma = jnp.dot(onehot, gt_ref[...], preferred_element_type=jnp.float32)
        feat = jnp.dot(onehot, et_ref[...], preferred_element_type=jnp.float32)
        a = gamma * w[None, :]
        c = gamma * bias[None, :] + feat
        pp = pp_ref[g]
        out_ref[g] = pp[:, :, None] * a[None] + c[None]
    valid_ref[...] = jnp.broadcast_to((ids != MASK_ID)[:, None, :], (G, T, D))


def _tc2_body(tok_ref, val_ref, ids_ref, pp_ref, ff_ref, gg_ref, w_ref, b_ref,
              out_ref, valid_ref):
    i = pl.program_id(0)
    ids = ids_ref[pl.ds((i + HBLK) * G, G), :]
    w = w_ref[0, :]
    bias = b_ref[0, :]
    for g in range(G):
        gamma = gg_ref[g]
        feat = ff_ref[g]
        a = gamma * w[None, :]
        c = gamma * bias[None, :] + feat
        pp = pp_ref[g]
        out_ref[g] = pp[:, :, None] * a[None] + c[None]
    valid_ref[...] = jnp.broadcast_to((ids != MASK_ID)[:, None, :], (G, T, D))


def kernel(proprio, dof_ids, embed_table, gamma_table, W, b):
    dof_ids = dof_ids.astype(jnp.int32)
    idx2 = dof_ids[H:].reshape(HD)
    feat2, gamma2 = _sc_gather(idx2, embed_table, gamma_table)
    feat2 = feat2.reshape(H, D, E)
    gamma2 = gamma2.reshape(H, D, E)
    b2 = b.reshape(1, E)

    out_shapes = [
        jax.ShapeDtypeStruct((B, T, D, E), jnp.float32),
        jax.ShapeDtypeStruct((B, T, D), jnp.bool_),
    ]
    tok1, val1 = pl.pallas_call(
        _tc1_body,
        grid=(HBLK,),
        in_specs=[
            pl.BlockSpec((B, D), lambda i: (0, 0)),
            pl.BlockSpec((G, T, D), lambda i: (i, 0, 0)),
            pl.BlockSpec((V, E), lambda i: (0, 0)),
            pl.BlockSpec((V, E), lambda i: (0, 0)),
            pl.BlockSpec((1, E), lambda i: (0, 0)),
            pl.BlockSpec((1, E), lambda i: (0, 0)),
        ],
        out_specs=[
            pl.BlockSpec((G, T, D, E), lambda i: (i, 0, 0, 0)),
            pl.BlockSpec((G, T, D), lambda i: (i, 0, 0)),
        ],
        out_shape=out_shapes,
    )(dof_ids, proprio, embed_table, gamma_table, W, b2)

    tokens, valid = pl.pallas_call(
        _tc2_body,
        grid=(HBLK,),
        in_specs=[
            pl.BlockSpec(memory_space=pl.ANY),
            pl.BlockSpec(memory_space=pl.ANY),
            pl.BlockSpec((B, D), lambda i: (0, 0)),
            pl.BlockSpec((G, T, D), lambda i: (i + HBLK, 0, 0)),
            pl.BlockSpec((G, D, E), lambda i: (i, 0, 0)),
            pl.BlockSpec((G, D, E), lambda i: (i, 0, 0)),
            pl.BlockSpec((1, E), lambda i: (0, 0)),
            pl.BlockSpec((1, E), lambda i: (0, 0)),
        ],
        out_specs=[
            pl.BlockSpec((G, T, D, E), lambda i: (i + HBLK, 0, 0, 0)),
            pl.BlockSpec((G, T, D), lambda i: (i + HBLK, 0, 0)),
        ],
        out_shape=out_shapes,
        input_output_aliases={0: 0, 1: 1},
    )(tok1, val1, dof_ids, proprio, feat2, gamma2, W, b2)
    return tokens, valid


# speedup vs baseline: 1.5198x; 1.3612x over previous
import functools

import jax
import jax.numpy as jnp
from jax.experimental import pallas as pl

B, T, D, E, V = 128, 32, 32, 256, 32
MASK_ID = 0


G = 8
TS = 16
NT = T // TS


def _tc_body(ids_ref, pp_ref, et_ref, gt_ref, w_ref, b_ref, out_ref, valid_ref):
    i = pl.program_id(0)
    ids = ids_ref[pl.ds(i * G, G), :]
    w = w_ref[0, :]
    bias = b_ref[0, :]
    for g in range(G):
        idsg = ids[g, :]
        onehot = (idsg[:, None] == jax.lax.broadcasted_iota(jnp.int32, (D, V), 1)
                  ).astype(jnp.float32)
        gamma = jnp.dot(onehot, gt_ref[...], preferred_element_type=jnp.float32)
        feat = jnp.dot(onehot, et_ref[...], preferred_element_type=jnp.float32)
        a = gamma * w[None, :]
        c = gamma * bias[None, :] + feat
        pp = pp_ref[g]
        out_ref[g] = pp[:, :, None] * a[None] + c[None]
    valid_ref[...] = jnp.broadcast_to((ids != MASK_ID)[:, None, :], (G, TS, D))


def kernel(proprio, dof_ids, embed_table, gamma_table, W, b):
    dof_ids = dof_ids.astype(jnp.int32)
    b2 = b.reshape(1, E)
    grid = (B // G, NT)
    tokens, valid = pl.pallas_call(
        _tc_body,
        grid=grid,
        in_specs=[
            pl.BlockSpec((B, D), lambda i, j: (0, 0)),
            pl.BlockSpec((G, TS, D), lambda i, j: (i, j, 0)),
            pl.BlockSpec((V, E), lambda i, j: (0, 0)),
            pl.BlockSpec((V, E), lambda i, j: (0, 0)),
            pl.BlockSpec((1, E), lambda i, j: (0, 0)),
            pl.BlockSpec((1, E), lambda i, j: (0, 0)),
        ],
        out_specs=[
            pl.BlockSpec((G, TS, D, E), lambda i, j: (i, j, 0, 0)),
            pl.BlockSpec((G, TS, D), lambda i, j: (i, j, 0)),
        ],
        out_shape=[
            jax.ShapeDtypeStruct((B, T, D, E), jnp.float32),
            jax.ShapeDtypeStruct((B, T, D), jnp.bool_),
        ],
    )(dof_ids, proprio, embed_table, gamma_table, W, b2)
    return tokens, valid


# speedup vs baseline: 1.6280x; 1.0712x over previous
import functools

import jax
import jax.numpy as jnp
from jax.experimental import pallas as pl

B, T, D, E, V = 128, 32, 32, 256, 32
MASK_ID = 0


G = 8


def _tc_body(ids_ref, pp_ref, et_ref, gt_ref, w_ref, b_ref, out_ref, valid_ref):
    i = pl.program_id(0)
    ids = ids_ref[pl.ds(i * G, G), :]
    w = w_ref[0, :]
    bias = b_ref[0, :]
    for g in range(G):
        idsg = ids[g, :]
        onehot = (idsg[:, None] == jax.lax.broadcasted_iota(jnp.int32, (D, V), 1)
                  ).astype(jnp.float32)
        gamma = jnp.dot(onehot, gt_ref[...], preferred_element_type=jnp.float32)
        feat = jnp.dot(onehot, et_ref[...], preferred_element_type=jnp.float32)
        a = gamma * w[None, :]
        c = gamma * bias[None, :] + feat
        pp = pp_ref[g]
        out_ref[g] = pp[:, :, None] * a[None] + c[None]
    valid_ref[...] = jnp.broadcast_to((ids != MASK_ID)[:, None, :], (G, T, D))


def kernel(proprio, dof_ids, embed_table, gamma_table, W, b):
    dof_ids = dof_ids.astype(jnp.int32)
    b2 = b.reshape(1, E)
    grid = (B // G,)
    tokens, valid = pl.pallas_call(
        _tc_body,
        grid=grid,
        in_specs=[
            pl.BlockSpec((B, D), lambda i: (0, 0)),
            pl.BlockSpec((G, T, D), lambda i: (i, 0, 0)),
            pl.BlockSpec((V, E), lambda i: (0, 0)),
            pl.BlockSpec((V, E), lambda i: (0, 0)),
            pl.BlockSpec((1, E), lambda i: (0, 0)),
            pl.BlockSpec((1, E), lambda i: (0, 0)),
        ],
        out_specs=[
            pl.BlockSpec((G, T, D, E), lambda i: (i, 0, 0, 0)),
            pl.BlockSpec((G, T, D), lambda i: (i, 0, 0)),
        ],
        out_shape=[
            jax.ShapeDtypeStruct((B, T, D, E), jnp.float32),
            jax.ShapeDtypeStruct((B, T, D), jnp.bool_),
        ],
    )(dof_ids, proprio, embed_table, gamma_table, W, b2)
    return tokens, valid


# speedup vs baseline: 1.6329x; 1.0030x over previous
import jax
import jax.numpy as jnp
from jax.experimental import pallas as pl

B, T, D, E, V = 128, 32, 32, 256, 32
MASK_ID = 0

G = 8
TC = 8


def _tc_body(idsf_ref, ids_ref, pp_ref, et_ref, gt_ref, w_ref, b_ref,
             out_ref, valid_ref):
    i = pl.program_id(0)
    idsf = idsf_ref[pl.ds(i * G * D, G * D), :]
    onehot = (idsf == jax.lax.broadcasted_iota(jnp.int32, (G * D, V), 1)
              ).astype(jnp.float32)
    gamma = jnp.dot(onehot, gt_ref[...], preferred_element_type=jnp.float32)
    feat = jnp.dot(onehot, et_ref[...], preferred_element_type=jnp.float32)
    w = w_ref[0, :]
    bias = b_ref[0, :]
    a_all = gamma * w[None, :]
    c_all = gamma * bias[None, :] + feat
    for g in range(G):
        a = a_all[g * D:(g + 1) * D, :]
        c = c_all[g * D:(g + 1) * D, :]
        for t0 in range(0, T, TC):
            pp = pp_ref[g, pl.ds(t0, TC), :]
            out_ref[g, pl.ds(t0, TC)] = pp[:, :, None] * a[None] + c[None]
    ids = ids_ref[pl.ds(i * G, G), :]
    valid_ref[...] = jnp.broadcast_to((ids != MASK_ID)[:, None, :], (G, T, D))


def kernel(proprio, dof_ids, embed_table, gamma_table, W, b):
    dof_ids = dof_ids.astype(jnp.int32)
    idsf = dof_ids.reshape(B * D, 1)
    b2 = b.reshape(1, E)
    grid = (B // G,)
    tokens, valid = pl.pallas_call(
        _tc_body,
        grid=grid,
        in_specs=[
            pl.BlockSpec((B * D, 1), lambda i: (0, 0)),
            pl.BlockSpec((B, D), lambda i: (0, 0)),
            pl.BlockSpec((G, T, D), lambda i: (i, 0, 0)),
            pl.BlockSpec((V, E), lambda i: (0, 0)),
            pl.BlockSpec((V, E), lambda i: (0, 0)),
            pl.BlockSpec((1, E), lambda i: (0, 0)),
            pl.BlockSpec((1, E), lambda i: (0, 0)),
        ],
        out_specs=[
            pl.BlockSpec((G, T, D, E), lambda i: (i, 0, 0, 0)),
            pl.BlockSpec((G, T, D), lambda i: (i, 0, 0)),
        ],
        out_shape=[
            jax.ShapeDtypeStruct((B, T, D, E), jnp.float32),
            jax.ShapeDtypeStruct((B, T, D), jnp.bool_),
        ],
    )(idsf, dof_ids, proprio, embed_table, gamma_table, W, b2)
    return tokens, valid
